# Initial kernel scaffold; baseline (speedup 1.0000x reference)
import jax
import jax.numpy as jnp
from jax.experimental import pallas as pl


def kernel(target_batch, context_batch, embedding_w):
    raise NotImplementedError("write your pallas kernel here")


# speedup vs baseline: 1.5009x; 1.5009x over previous
import functools

import jax
import jax.numpy as jnp
from jax import lax
from jax.experimental import pallas as pl
from jax.experimental.pallas import tpu as pltpu
from jax.experimental.pallas import tpu_sc as plsc

VOCAB = 100000
EMBED = 128
BATCH = 4096

_info = plsc.get_sparse_core_info()
_NC, _NS = _info.num_cores, _info.num_subcores
_NW = _NC * _NS
_BPW = BATCH // _NW


def _make_lookup_kernel():
    mesh = plsc.VectorSubcoreMesh(core_axis_name="c", subcore_axis_name="s")

    @functools.partial(
        pl.kernel,
        mesh=mesh,
        out_type=[
            jax.ShapeDtypeStruct((BATCH, EMBED), jnp.float32),
            jax.ShapeDtypeStruct((BATCH, EMBED), jnp.float32),
        ],
        scratch_types=[
            pltpu.VMEM((_BPW,), jnp.int32),
            pltpu.VMEM((_BPW,), jnp.int32),
            pltpu.VMEM((_BPW, EMBED), jnp.float32),
            pltpu.VMEM((_BPW, EMBED), jnp.float32),
            pltpu.SemaphoreType.DMA,
            pltpu.SemaphoreType.DMA,
        ],
    )
    def lookup(tgt_hbm, ctx_hbm, table_hbm, out_t_hbm, out_c_hbm,
               idx_t, idx_c, rows_t, rows_c, sem_t, sem_c):
        wid = lax.axis_index("s") * _NC + lax.axis_index("c")
        base = wid * _BPW
        pltpu.sync_copy(tgt_hbm.at[pl.ds(base, _BPW)], idx_t)
        pltpu.sync_copy(ctx_hbm.at[pl.ds(base, _BPW)], idx_c)
        cp_t = pltpu.async_copy(table_hbm.at[idx_t], rows_t, sem_t)
        cp_c = pltpu.async_copy(table_hbm.at[idx_c], rows_c, sem_c)
        cp_t.wait()
        pltpu.sync_copy(rows_t, out_t_hbm.at[pl.ds(base, _BPW)])
        cp_c.wait()
        pltpu.sync_copy(rows_c, out_c_hbm.at[pl.ds(base, _BPW)])

    return lookup


_lookup = _make_lookup_kernel()


@jax.jit
def kernel(target_batch, context_batch, embedding_w):
    embedded, embedded_context = _lookup(target_batch, context_batch, embedding_w)
    return embedded, embedded_context


# speedup vs baseline: 1.5333x; 1.0216x over previous
import functools

import jax
import jax.numpy as jnp
from jax import lax
from jax.experimental import pallas as pl
from jax.experimental.pallas import tpu as pltpu
from jax.experimental.pallas import tpu_sc as plsc

VOCAB = 100000
EMBED = 128
BATCH = 4096

_info = plsc.get_sparse_core_info()
_NC, _NS = _info.num_cores, _info.num_subcores
_NW = _NC * _NS
_BPW = BATCH // _NW


def _make_lookup_kernel():
    mesh = plsc.VectorSubcoreMesh(core_axis_name="c", subcore_axis_name="s")

    @functools.partial(
        pl.kernel,
        mesh=mesh,
        out_type=[
            jax.ShapeDtypeStruct((BATCH, EMBED), jnp.float32),
            jax.ShapeDtypeStruct((BATCH, EMBED), jnp.float32),
        ],
        scratch_types=[
            pltpu.VMEM((_BPW,), jnp.int32),
            pltpu.VMEM((_BPW,), jnp.int32),
            pltpu.VMEM((_BPW, EMBED), jnp.float32),
            pltpu.VMEM((_BPW, EMBED), jnp.float32),
            pltpu.SemaphoreType.DMA,
            pltpu.SemaphoreType.DMA,
            pltpu.SemaphoreType.DMA,
            pltpu.SemaphoreType.DMA,
            pltpu.SemaphoreType.DMA,
            pltpu.SemaphoreType.DMA,
            pltpu.SemaphoreType.DMA,
            pltpu.SemaphoreType.DMA,
            pltpu.SemaphoreType.DMA,
            pltpu.SemaphoreType.DMA,
        ],
    )
    def lookup(tgt_hbm, ctx_hbm, table_hbm, out_t_hbm, out_c_hbm,
               idx_t, idx_c, rows_t, rows_c,
               sem_it, sem_ic, sem_gt0, sem_gt1, sem_gc0, sem_gc1,
               sem_wt0, sem_wt1, sem_wc0, sem_wc1):
        wid = lax.axis_index("s") * _NC + lax.axis_index("c")
        base = wid * _BPW
        half = _BPW // 2
        ci_t = pltpu.async_copy(tgt_hbm.at[pl.ds(base, _BPW)], idx_t, sem_it)
        ci_c = pltpu.async_copy(ctx_hbm.at[pl.ds(base, _BPW)], idx_c, sem_ic)
        ci_t.wait()
        g_t0 = pltpu.async_copy(
            table_hbm.at[idx_t.at[pl.ds(0, half)]], rows_t.at[pl.ds(0, half)], sem_gt0)
        g_t1 = pltpu.async_copy(
            table_hbm.at[idx_t.at[pl.ds(half, half)]], rows_t.at[pl.ds(half, half)], sem_gt1)
        ci_c.wait()
        g_c0 = pltpu.async_copy(
            table_hbm.at[idx_c.at[pl.ds(0, half)]], rows_c.at[pl.ds(0, half)], sem_gc0)
        g_c1 = pltpu.async_copy(
            table_hbm.at[idx_c.at[pl.ds(half, half)]], rows_c.at[pl.ds(half, half)], sem_gc1)
        g_t0.wait()
        w_t0 = pltpu.async_copy(
            rows_t.at[pl.ds(0, half)], out_t_hbm.at[pl.ds(base, half)], sem_wt0)
        g_c0.wait()
        w_c0 = pltpu.async_copy(
            rows_c.at[pl.ds(0, half)], out_c_hbm.at[pl.ds(base, half)], sem_wc0)
        g_t1.wait()
        w_t1 = pltpu.async_copy(
            rows_t.at[pl.ds(half, half)], out_t_hbm.at[pl.ds(base + half, half)], sem_wt1)
        g_c1.wait()
        w_c1 = pltpu.async_copy(
            rows_c.at[pl.ds(half, half)], out_c_hbm.at[pl.ds(base + half, half)], sem_wc1)
        w_t0.wait()
        w_c0.wait()
        w_t1.wait()
        w_c1.wait()

    return lookup


_lookup = _make_lookup_kernel()


@jax.jit
def kernel(target_batch, context_batch, embedding_w):
    embedded, embedded_context = _lookup(target_batch, context_batch, embedding_w)
    return embedded, embedded_context
